# Initial kernel scaffold; baseline (speedup 1.0000x reference)
import jax
import jax.numpy as jnp
from jax.experimental import pallas as pl


def kernel(x, edge_index, W0, b0, W1, b1, eps, Wa, ba, g1, be1, Wb, bb, g2, be2):
    raise NotImplementedError("write your pallas kernel here")


# speedup vs baseline: 3.8419x; 3.8419x over previous
import functools

import jax
import jax.numpy as jnp
from jax import lax
from jax.experimental import pallas as pl
from jax.experimental.pallas import tpu as pltpu
from jax.experimental.pallas import tpu_sc as plsc

N = 10000
E = 320000
IN = 128
H = 256
OUT = 128
L = 3
HH = H // 2

NC = 2
NS = 16

CHUNK = 128
EPT_RAW = E // NS
NCHUNK = -(-EPT_RAW // CHUNK)
EPT = NCHUNK * CHUNK
EPAD = NS * EPT
RPT = 632
AGG_ROWS = N + 8

BN = 1000


def _sc_mesh():
    return plsc.VectorSubcoreMesh(core_axis_name="c", subcore_axis_name="s")


@functools.partial(
    pl.kernel,
    out_type=jax.ShapeDtypeStruct((2, N, HH), jnp.float32),
    mesh=_sc_mesh(),
    scratch_types=[
        pltpu.VMEM((CHUNK,), jnp.int32),
        pltpu.VMEM((CHUNK,), jnp.int32),
        pltpu.VMEM((CHUNK, HH), jnp.float32),
        pltpu.VMEM_SHARED((AGG_ROWS, HH), jnp.float32),
        pltpu.SemaphoreType.DMA,
    ],
)
def _sc_segsum(hflat, hs, src2, dst, out, sidx, didx, rows, aggs, sem):
    c = lax.axis_index("c")
    s = lax.axis_index("s")
    r0 = pl.multiple_of(jnp.minimum(s * RPT, N - RPT), 8)

    pltpu.sync_copy(hs.at[c, pl.ds(r0, RPT)], aggs.at[pl.ds(r0, RPT)])
    plsc.subcore_barrier()

    base = s * EPT

    def body(i, carry):
        off = base + i * CHUNK
        pltpu.sync_copy(src2.at[c, pl.ds(off, CHUNK)], sidx)
        pltpu.sync_copy(dst.at[pl.ds(off, CHUNK)], didx)
        pltpu.async_copy(hflat.at[sidx], rows, sem).wait()
        pltpu.sync_copy(rows, aggs.at[didx], add=True)
        return carry

    lax.fori_loop(0, NCHUNK, body, 0)
    plsc.subcore_barrier()

    pltpu.sync_copy(aggs.at[pl.ds(r0, RPT)], out.at[c, pl.ds(r0, RPT)])


def _tc_init_body(x_ref, w_ref, b_ref, sc_ref, hg_ref, hs_ref):
    h = jnp.dot(x_ref[...], w_ref[...], preferred_element_type=jnp.float32)
    h = jnp.maximum(h + b_ref[...], 0.0)
    hg_ref[0] = h[:, :HH]
    hg_ref[1] = h[:, HH:]
    s = sc_ref[0]
    hs_ref[0] = s * h[:, :HH]
    hs_ref[1] = s * h[:, HH:]


def _tc_layer_body(m_ref, wa_ref, ba_ref, wb_ref, bb_ref, sc_ref, hg_ref, hs_ref):
    m = jnp.concatenate([m_ref[0], m_ref[1]], axis=1)
    t = jnp.dot(m, wa_ref[...], preferred_element_type=jnp.float32)
    t = jnp.maximum(t + ba_ref[...], 0.0)
    h = jnp.dot(t, wb_ref[...], preferred_element_type=jnp.float32)
    h = jnp.maximum(h + bb_ref[...], 0.0)
    hg_ref[0] = h[:, :HH]
    hg_ref[1] = h[:, HH:]
    s = sc_ref[0]
    hs_ref[0] = s * h[:, :HH]
    hs_ref[1] = s * h[:, HH:]


def _tc_last_body(m_ref, wa_ref, ba_ref, wb_ref, bb_ref, w1_ref, b1_ref, o_ref):
    m = jnp.concatenate([m_ref[0], m_ref[1]], axis=1)
    t = jnp.dot(m, wa_ref[...], preferred_element_type=jnp.float32)
    t = jnp.maximum(t + ba_ref[...], 0.0)
    h = jnp.dot(t, wb_ref[...], preferred_element_type=jnp.float32)
    h = jnp.maximum(h + bb_ref[...], 0.0)
    o_ref[...] = jnp.dot(h, w1_ref[...], preferred_element_type=jnp.float32) + b1_ref[...]


def _full_spec(shape):
    return pl.BlockSpec(shape, lambda i: (0,) * len(shape))


_SCALE_SPEC = pl.BlockSpec(memory_space=pltpu.SMEM)


_PAIR_SPEC = pl.BlockSpec((2, BN, HH), lambda i: (0, i, 0))
_GRID = N // BN


def _tc_init(x, w0, b0, scale):
    return pl.pallas_call(
        _tc_init_body,
        grid=(_GRID,),
        in_specs=[
            pl.BlockSpec((BN, IN), lambda i: (i, 0)),
            _full_spec((IN, H)),
            _full_spec((1, H)),
            _SCALE_SPEC,
        ],
        out_specs=[_PAIR_SPEC, _PAIR_SPEC],
        out_shape=[
            jax.ShapeDtypeStruct((2, N, HH), jnp.float32),
            jax.ShapeDtypeStruct((2, N, HH), jnp.float32),
        ],
    )(x, w0, b0.reshape(1, H), scale)


def _tc_layer(m2, wa, ba, wb, bb, scale):
    return pl.pallas_call(
        _tc_layer_body,
        grid=(_GRID,),
        in_specs=[
            _PAIR_SPEC,
            _full_spec((H, H)),
            _full_spec((1, H)),
            _full_spec((H, H)),
            _full_spec((1, H)),
            _SCALE_SPEC,
        ],
        out_specs=[_PAIR_SPEC, _PAIR_SPEC],
        out_shape=[
            jax.ShapeDtypeStruct((2, N, HH), jnp.float32),
            jax.ShapeDtypeStruct((2, N, HH), jnp.float32),
        ],
    )(m2, wa, ba, wb, bb, scale)


def _tc_last(m2, wa, ba, wb, bb, w1, b1):
    return pl.pallas_call(
        _tc_last_body,
        grid=(_GRID,),
        in_specs=[
            _PAIR_SPEC,
            _full_spec((H, H)),
            _full_spec((1, H)),
            _full_spec((H, H)),
            _full_spec((1, H)),
            _full_spec((H, OUT)),
            _full_spec((1, OUT)),
        ],
        out_specs=pl.BlockSpec((BN, OUT), lambda i: (i, 0)),
        out_shape=jax.ShapeDtypeStruct((N, OUT), jnp.float32),
    )(m2, wa, ba, wb, bb, w1, b1.reshape(1, OUT))


def kernel(x, edge_index, W0, b0, W1, b1, eps, Wa, ba, g1, be1, Wb, bb, g2, be2):
    src = edge_index[0].reshape(NS, EPT_RAW)
    dstr = edge_index[1].reshape(NS, EPT_RAW)
    pad = EPT - EPT_RAW
    src_p = jnp.pad(src, ((0, 0), (0, pad))).reshape(-1)
    dst_p = jnp.pad(dstr, ((0, 0), (0, pad)), constant_values=N).reshape(-1)
    src2 = jnp.stack([src_p, src_p + N])

    kbn = 1.0 / jnp.sqrt(jnp.float32(1.0 + 1e-5))
    a1 = g1 * kbn
    a2 = g2 * kbn
    wa_f = Wa * a1[:, None, :]
    ba_f = (ba * a1 + be1).reshape(L, 1, H)
    wb_f = Wb * a2[:, None, :]
    bb_f = (bb * a2 + be2).reshape(L, 1, H)
    scales = (1.0 + eps).astype(jnp.float32)

    hg, hs = _tc_init(x, W0, b0, scales[0:1])
    for i in range(L):
        m2 = _sc_segsum(hg.reshape(2 * N, HH), hs, src2, dst_p)
        if i < L - 1:
            hg, hs = _tc_layer(m2, wa_f[i], ba_f[i], wb_f[i], bb_f[i],
                               scales[i + 1:i + 2])
        else:
            return _tc_last(m2, wa_f[i], ba_f[i], wb_f[i], bb_f[i], W1, b1)
